# baseline (speedup 1.0000x reference)
import math

import jax
import jax.numpy as jnp
from jax.experimental import pallas as pl
from jax.experimental.pallas import tpu as pltpu

H = 512
M = 65536
BATCH = 8
TM = 1024
NT = M // TM
SCALE = 1.0 / math.sqrt(H)


def _pass1_body(h_ref, wq_ref, bq_ref, wk_ref, bk_ref, wv_ref, bv_ref,
                age_ref, k_ref,
                nk_ref, probs_ref, updk_ref, updv_ref, idx_ref, counts_ref,
                maxsc_ref, usage_ref, age_out_ref,
                q_scr, s_scr):
    i = pl.program_id(0)

    @pl.when(i == 0)
    def _():
        h = h_ref[...]
        q_scr[...] = (jnp.dot(h, wq_ref[...].T,
                              preferred_element_type=jnp.float32)
                      + bq_ref[...]) * SCALE
        updk_ref[...] = jnp.dot(h[0:1], wk_ref[...].T,
                                preferred_element_type=jnp.float32) + bk_ref[...]
        updv_ref[...] = jnp.dot(h[0:1], wv_ref[...].T,
                                preferred_element_type=jnp.float32) + bv_ref[...]

    k = k_ref[...]
    nk_ref[...] = k
    s_scr[:, pl.ds(i * TM, TM)] = jnp.dot(q_scr[...], k.T,
                                          preferred_element_type=jnp.float32)

    @pl.when(i == NT - 1)
    def _():
        s = s_scr[...]
        m = jnp.max(s, axis=1, keepdims=True)
        e = jnp.exp(s - m)
        z = jnp.sum(e, axis=1, keepdims=True)
        p = e / z
        probs_ref[...] = p
        imp = jnp.sum(p, axis=0, keepdims=True)
        age1 = age_ref[...] + 1.0
        t = age1 + (1.0 - imp)
        maxv = jnp.max(t)
        iota = jax.lax.broadcasted_iota(jnp.int32, (1, M), 1)
        idx = jnp.min(jnp.where(t == maxv, iota, M))
        idx_ref[...] = jnp.full((1, 1), idx, jnp.int32)
        counts_ref[...] = jnp.sum((p > 0.01).astype(jnp.int32), axis=0,
                                  keepdims=True)
        maxsc_ref[...] = jnp.full((1, 1), jnp.mean(m), jnp.float32)
        new_age = jnp.where(iota == idx, 0.0, age1)
        age_out_ref[...] = new_age
        usage_ref[...] = jnp.full(
            (1, 1), jnp.mean((new_age > 0.0).astype(jnp.float32)), jnp.float32)


def _pass2_body(p_ref, v_ref, updv_ref, wo_ref, bo_ref, idx_ref,
                nv_ref, out_ref, acc_scr):
    i = pl.program_id(0)
    v = v_ref[...]

    @pl.when(i == 0)
    def _():
        acc_scr[...] = jnp.zeros_like(acc_scr)

    acc_scr[...] += jnp.dot(p_ref[...], v, preferred_element_type=jnp.float32)
    rows = jax.lax.broadcasted_iota(jnp.int32, (TM, 1), 0) + i * TM
    nv_ref[...] = jnp.where(rows == idx_ref[0], updv_ref[...], v)

    @pl.when(i == NT - 1)
    def _():
        out_ref[...] = jnp.dot(acc_scr[...], wo_ref[...].T,
                               preferred_element_type=jnp.float32) + bo_ref[...]


def _scatter_body(idx_ref, big_ref, upd_ref, out_ref):
    del idx_ref, big_ref
    out_ref[...] = upd_ref[...]


def kernel(hidden_states, Wq, bq, Wk, bk, Wv, bv, Wo, bo,
           memory_keys, memory_values, memory_age):
    f32 = jnp.float32
    h = hidden_states.reshape(BATCH, H)
    keys2 = memory_keys.reshape(M, H)
    vals2 = memory_values.reshape(M, H)
    bq2 = bq.reshape(1, H)
    bk2 = bk.reshape(1, H)
    bv2 = bv.reshape(1, H)
    bo2 = bo.reshape(1, H)

    def const(shape):
        return pl.BlockSpec(shape, lambda *_: (0,) * len(shape))

    nk, probs, updk, updv, idx11, counts, maxsc, usage, new_age = pl.pallas_call(
        _pass1_body,
        grid=(NT,),
        in_specs=[
            const((BATCH, H)),
            const((H, H)),
            const((1, H)),
            const((H, H)),
            const((1, H)),
            const((H, H)),
            const((1, H)),
            const((1, M)),
            pl.BlockSpec((TM, H), lambda i: (i, 0)),
        ],
        out_specs=[
            pl.BlockSpec((TM, H), lambda i: (i, 0)),
            const((BATCH, M)),
            const((1, H)),
            const((1, H)),
            const((1, 1)),
            const((1, M)),
            const((1, 1)),
            const((1, 1)),
            const((1, M)),
        ],
        out_shape=[
            jax.ShapeDtypeStruct((M, H), f32),
            jax.ShapeDtypeStruct((BATCH, M), f32),
            jax.ShapeDtypeStruct((1, H), f32),
            jax.ShapeDtypeStruct((1, H), f32),
            jax.ShapeDtypeStruct((1, 1), jnp.int32),
            jax.ShapeDtypeStruct((1, M), jnp.int32),
            jax.ShapeDtypeStruct((1, 1), f32),
            jax.ShapeDtypeStruct((1, 1), f32),
            jax.ShapeDtypeStruct((1, M), f32),
        ],
        scratch_shapes=[
            pltpu.VMEM((BATCH, H), f32),
            pltpu.VMEM((BATCH, M), f32),
        ],
        compiler_params=pltpu.CompilerParams(
            dimension_semantics=("arbitrary",)),
    )(h, Wq, bq2, Wk, bk2, Wv, bv2, memory_age, keys2)

    idx1 = idx11.reshape(1)

    nv, out = pl.pallas_call(
        _pass2_body,
        grid=(NT,),
        in_specs=[
            pl.BlockSpec((BATCH, TM), lambda i: (0, i)),
            pl.BlockSpec((TM, H), lambda i: (i, 0)),
            const((1, H)),
            const((H, H)),
            const((1, H)),
            pl.BlockSpec(memory_space=pltpu.SMEM),
        ],
        out_specs=[
            pl.BlockSpec((TM, H), lambda i: (i, 0)),
            const((BATCH, H)),
        ],
        out_shape=[
            jax.ShapeDtypeStruct((M, H), f32),
            jax.ShapeDtypeStruct((BATCH, H), f32),
        ],
        scratch_shapes=[pltpu.VMEM((BATCH, H), f32)],
        compiler_params=pltpu.CompilerParams(
            dimension_semantics=("arbitrary",)),
    )(probs, vals2, updv, Wo, bo2, idx1)

    nk3 = nk.reshape(M, 1, H)
    updk3 = updk.reshape(1, 1, H)
    nk_final = pl.pallas_call(
        _scatter_body,
        grid_spec=pltpu.PrefetchScalarGridSpec(
            num_scalar_prefetch=1,
            grid=(1,),
            in_specs=[
                pl.BlockSpec(memory_space=pl.ANY),
                pl.BlockSpec((1, 1, H), lambda i, idx_ref: (0, 0, 0)),
            ],
            out_specs=pl.BlockSpec((1, 1, H),
                                   lambda i, idx_ref: (idx_ref[0], 0, 0)),
        ),
        out_shape=jax.ShapeDtypeStruct((M, 1, H), f32),
        input_output_aliases={1: 0},
    )(idx1, nk3, updk3)

    output = out.reshape(BATCH, 1, H)
    max_scores = maxsc.reshape(())
    memory_usage = usage.reshape(())
    new_keys = nk_final.reshape(1, M, H)
    new_values = nv.reshape(1, M, H)
    return (output, counts, max_scores, memory_usage,
            new_keys, new_values, new_age)


# speedup vs baseline: 1.8925x; 1.8925x over previous
import math

import jax
import jax.numpy as jnp
from jax.experimental import pallas as pl
from jax.experimental.pallas import tpu as pltpu

H = 512
M = 65536
BATCH = 8
TM = 1024
NT = M // TM
SCALE = 1.0 / math.sqrt(H)


def _pass1_body(h_ref, wq_ref, bq_ref, wk_ref, bk_ref, wv_ref, bv_ref,
                age_ref, k_ref,
                nk_ref, probs_ref, updk_ref, updv_ref, idx_ref, counts_ref,
                maxsc_ref, usage_ref, age_out_ref,
                q_scr, s_scr):
    i = pl.program_id(0)

    @pl.when(i == 0)
    def _():
        h = h_ref[...]
        q_scr[...] = (jnp.dot(h, wq_ref[...].T,
                              preferred_element_type=jnp.float32)
                      + bq_ref[...]) * SCALE
        updk_ref[...] = jnp.dot(h[0:1], wk_ref[...].T,
                                preferred_element_type=jnp.float32) + bk_ref[...]
        updv_ref[...] = jnp.dot(h[0:1], wv_ref[...].T,
                                preferred_element_type=jnp.float32) + bv_ref[...]

    k = k_ref[0]
    nk_ref[...] = k_ref[...]
    s_scr[:, pl.ds(i * TM, TM)] = jnp.dot(q_scr[...], k.T,
                                          preferred_element_type=jnp.float32)

    @pl.when(i == NT - 1)
    def _():
        s = s_scr[...]
        m = jnp.max(s, axis=1, keepdims=True)
        e = jnp.exp(s - m)
        z = jnp.sum(e, axis=1, keepdims=True)
        p = e / z
        probs_ref[...] = p
        imp = jnp.sum(p, axis=0, keepdims=True)
        age1 = age_ref[...] + 1.0
        t = age1 + (1.0 - imp)
        maxv = jnp.max(t)
        iota = jax.lax.broadcasted_iota(jnp.int32, (1, M), 1)
        idx = jnp.min(jnp.where(t == maxv, iota, M))
        idx_ref[...] = jnp.full((1, 1), idx, jnp.int32)
        counts_ref[...] = jnp.sum((p > 0.01).astype(jnp.int32), axis=0,
                                  keepdims=True)
        maxsc_ref[...] = jnp.full((1, 1), jnp.mean(m), jnp.float32)
        new_age = jnp.where(iota == idx, 0.0, age1)
        age_out_ref[...] = new_age
        usage_ref[...] = jnp.full(
            (1, 1), jnp.mean((new_age > 0.0).astype(jnp.float32)), jnp.float32)


def _pass2_body(p_ref, v_ref, updv_ref, wo_ref, bo_ref, idx_ref,
                nv_ref, out_ref, acc_scr):
    i = pl.program_id(0)
    v = v_ref[0]

    @pl.when(i == 0)
    def _():
        acc_scr[...] = jnp.zeros_like(acc_scr)

    acc_scr[...] += jnp.dot(p_ref[...], v, preferred_element_type=jnp.float32)
    rows = jax.lax.broadcasted_iota(jnp.int32, (1, TM, 1), 1) + i * TM
    nv_ref[...] = jnp.where(rows == idx_ref[0], updv_ref[...][None], v_ref[...])

    @pl.when(i == NT - 1)
    def _():
        out_ref[...] = jnp.dot(acc_scr[...], wo_ref[...].T,
                               preferred_element_type=jnp.float32) + bo_ref[...]


def _scatter_body(idx_ref, rows_ref, upd_ref, out_ref):
    local = idx_ref[0] % 8
    lanes = jax.lax.broadcasted_iota(jnp.int32, (1, 8, 1), 1)
    out_ref[...] = jnp.where(lanes == local, upd_ref[...], rows_ref[...])


def kernel(hidden_states, Wq, bq, Wk, bk, Wv, bv, Wo, bo,
           memory_keys, memory_values, memory_age):
    f32 = jnp.float32
    h = hidden_states.reshape(BATCH, H)
    bq2 = bq.reshape(1, H)
    bk2 = bk.reshape(1, H)
    bv2 = bv.reshape(1, H)
    bo2 = bo.reshape(1, H)

    def const(shape):
        return pl.BlockSpec(shape, lambda *_: (0,) * len(shape))

    nk, probs, updk, updv, idx11, counts, maxsc, usage, new_age = pl.pallas_call(
        _pass1_body,
        grid=(NT,),
        in_specs=[
            const((BATCH, H)),
            const((H, H)),
            const((1, H)),
            const((H, H)),
            const((1, H)),
            const((H, H)),
            const((1, H)),
            const((1, M)),
            pl.BlockSpec((1, TM, H), lambda i: (0, i, 0)),
        ],
        out_specs=[
            pl.BlockSpec((1, TM, H), lambda i: (0, i, 0)),
            const((BATCH, M)),
            const((1, H)),
            const((1, H)),
            const((1, 1)),
            const((1, M)),
            const((1, 1)),
            const((1, 1)),
            const((1, M)),
        ],
        out_shape=[
            jax.ShapeDtypeStruct((1, M, H), f32),
            jax.ShapeDtypeStruct((BATCH, M), f32),
            jax.ShapeDtypeStruct((1, H), f32),
            jax.ShapeDtypeStruct((1, H), f32),
            jax.ShapeDtypeStruct((1, 1), jnp.int32),
            jax.ShapeDtypeStruct((1, M), jnp.int32),
            jax.ShapeDtypeStruct((1, 1), f32),
            jax.ShapeDtypeStruct((1, 1), f32),
            jax.ShapeDtypeStruct((1, M), f32),
        ],
        scratch_shapes=[
            pltpu.VMEM((BATCH, H), f32),
            pltpu.VMEM((BATCH, M), f32),
        ],
        compiler_params=pltpu.CompilerParams(
            dimension_semantics=("arbitrary",)),
    )(h, Wq, bq2, Wk, bk2, Wv, bv2, memory_age, memory_keys)

    idx1 = idx11.reshape(1)

    new_values, out = pl.pallas_call(
        _pass2_body,
        grid=(NT,),
        in_specs=[
            pl.BlockSpec((BATCH, TM), lambda i: (0, i)),
            pl.BlockSpec((1, TM, H), lambda i: (0, i, 0)),
            const((1, H)),
            const((H, H)),
            const((1, H)),
            pl.BlockSpec(memory_space=pltpu.SMEM),
        ],
        out_specs=[
            pl.BlockSpec((1, TM, H), lambda i: (0, i, 0)),
            const((BATCH, H)),
        ],
        out_shape=[
            jax.ShapeDtypeStruct((1, M, H), f32),
            jax.ShapeDtypeStruct((BATCH, H), f32),
        ],
        scratch_shapes=[pltpu.VMEM((BATCH, H), f32)],
        compiler_params=pltpu.CompilerParams(
            dimension_semantics=("arbitrary",)),
    )(probs, memory_values, updv, Wo, bo2, idx1)

    updk3 = updk.reshape(1, 1, H)
    new_keys = pl.pallas_call(
        _scatter_body,
        grid_spec=pltpu.PrefetchScalarGridSpec(
            num_scalar_prefetch=1,
            grid=(1,),
            in_specs=[
                pl.BlockSpec((1, 8, H), lambda i, idx_ref: (0, idx_ref[0] // 8, 0)),
                pl.BlockSpec((1, 1, H), lambda i, idx_ref: (0, 0, 0)),
            ],
            out_specs=pl.BlockSpec((1, 8, H),
                                   lambda i, idx_ref: (0, idx_ref[0] // 8, 0)),
        ),
        out_shape=jax.ShapeDtypeStruct((1, M, H), f32),
        input_output_aliases={1: 0},
    )(idx1, nk, updk3)

    output = out.reshape(BATCH, 1, H)
    max_scores = maxsc.reshape(())
    memory_usage = usage.reshape(())
    return (output, counts, max_scores, memory_usage,
            new_keys, new_values, new_age)
